# Initial kernel scaffold; baseline (speedup 1.0000x reference)
import jax
import jax.numpy as jnp
from jax.experimental import pallas as pl


def kernel(user_emb, item_emb, norm_adj_indices, norm_adj_values):
    raise NotImplementedError("write your pallas kernel here")


# speedup vs baseline: 10.8213x; 10.8213x over previous
import functools

import jax
import jax.numpy as jnp
from jax import lax
from jax.experimental import pallas as pl
from jax.experimental.pallas import tpu as pltpu
from jax.experimental.pallas import tpu_sc as plsc

N_USERS = 50000
N_ITEMS = 50000
D = 32
N = N_USERS + N_ITEMS

NC = 2
NS = 16
LANES = 16

BLK = 128
CHUNK_ROWS = 8
CHUNK_E = BLK * CHUNK_ROWS

N_ACC = 100096
ROWS_PER_TILE_OUT = N_ACC // NS


def _sc_body(table, src, dst, val, out, sidx_v, didx_v, val_v, rows_v,
             acc, sem, *, chunks_per_tile):
  c = lax.axis_index("c")
  s = lax.axis_index("s")

  def zero_rows(i, _):
    rows_v[i, :] = jnp.zeros((LANES,), jnp.float32)
    return 0
  lax.fori_loop(0, CHUNK_E, zero_rows, 0, unroll=4)
  zbase = s * ROWS_PER_TILE_OUT
  for r in range(ROWS_PER_TILE_OUT // CHUNK_E):
    pltpu.sync_copy(rows_v, acc.at[pl.ds(zbase + r * CHUNK_E, CHUNK_E)])
  rem = ROWS_PER_TILE_OUT % CHUNK_E
  if rem:
    pltpu.sync_copy(
        rows_v.at[pl.ds(0, rem)],
        acc.at[pl.ds(zbase + (ROWS_PER_TILE_OUT // CHUNK_E) * CHUNK_E, rem)])
  plsc.subcore_barrier()

  rows_per_tile = chunks_per_tile * CHUNK_ROWS

  def do_chunk(g, _):
    row0 = s * rows_per_tile + g * CHUNK_ROWS
    pltpu.sync_copy(src.at[pl.ds(row0, CHUNK_ROWS)], sidx_v)
    pltpu.sync_copy(dst.at[pl.ds(row0, CHUNK_ROWS)], didx_v)
    pltpu.sync_copy(val.at[pl.ds(row0, CHUNK_ROWS)], val_v)

    def fix_idx(i, _):
      for j in range(BLK // LANES):
        sl = pl.ds(j * LANES, LANES)
        sidx_v[i, sl] = sidx_v[i, sl] * 2 + c
      return 0
    lax.fori_loop(0, CHUNK_ROWS, fix_idx, 0)

    descs = [pltpu.async_copy(table.at[sidx_v.at[j]],
                              rows_v.at[pl.ds(j * BLK, BLK)], sem)
             for j in range(CHUNK_ROWS)]
    for d in descs:
      d.wait()

    def scale_row(i, _):
      def scale_blk(kb, _):
        base = kb * LANES
        vvec = val_v[i, pl.ds(base, LANES)]
        for u in range(LANES):
          e = i * BLK + base + u
          rows_v[e, :] = rows_v[e, :] * vvec[u]
        return 0
      lax.fori_loop(0, BLK // LANES, scale_blk, 0)
      return 0
    lax.fori_loop(0, CHUNK_ROWS, scale_row, 0)

    for j in range(CHUNK_ROWS):
      pltpu.sync_copy(rows_v.at[pl.ds(j * BLK, BLK)],
                      acc.at[didx_v.at[j]], add=True)
    return 0

  lax.fori_loop(0, chunks_per_tile, do_chunk, 0)
  plsc.subcore_barrier()

  pltpu.sync_copy(acc.at[pl.ds(s * ROWS_PER_TILE_OUT, ROWS_PER_TILE_OUT)],
                  out.at[c, pl.ds(s * ROWS_PER_TILE_OUT, ROWS_PER_TILE_OUT)])


@jax.jit
def kernel(user_emb, item_emb, norm_adj_indices, norm_adj_values):
  e = norm_adj_indices.shape[1]
  per_unit = NS * CHUNK_E
  e_pad = ((e + per_unit - 1) // per_unit) * per_unit
  pad = e_pad - e
  chunks_per_tile = e_pad // per_unit

  all_emb = jnp.concatenate([user_emb, item_emb], axis=0)
  table = all_emb.reshape(2 * N, D // 2)

  dst = norm_adj_indices[0].astype(jnp.int32)
  src = norm_adj_indices[1].astype(jnp.int32)
  val = norm_adj_values
  if pad:
    zi = jnp.zeros((pad,), jnp.int32)
    dst = jnp.concatenate([dst, zi])
    src = jnp.concatenate([src, zi])
    val = jnp.concatenate([val, jnp.zeros((pad,), val.dtype)])
  src2 = src.reshape(e_pad // BLK, BLK)
  dst2 = dst.reshape(e_pad // BLK, BLK)
  val2 = val.reshape(e_pad // BLK, BLK)

  mesh = plsc.VectorSubcoreMesh(core_axis_name="c", subcore_axis_name="s")
  out = pl.kernel(
      functools.partial(_sc_body, chunks_per_tile=chunks_per_tile),
      out_type=jax.ShapeDtypeStruct((NC, N_ACC, D // 2), jnp.float32),
      mesh=mesh,
      compiler_params=pltpu.CompilerParams(use_tc_tiling_on_sc=False),
      scratch_types=[
          pltpu.VMEM((CHUNK_ROWS, BLK), jnp.int32),
          pltpu.VMEM((CHUNK_ROWS, BLK), jnp.int32),
          pltpu.VMEM((CHUNK_ROWS, BLK), jnp.float32),
          pltpu.VMEM((CHUNK_E, LANES), jnp.float32),
          pltpu.VMEM_SHARED((N_ACC, D // 2), jnp.float32),
          pltpu.SemaphoreType.DMA,
      ],
  )(table, src2, dst2, val2)

  result = jnp.concatenate([out[0, :N], out[1, :N]], axis=1)
  return (result[:N_USERS], result[N_USERS:])


# speedup vs baseline: 16.5297x; 1.5275x over previous
import functools

import jax
import jax.numpy as jnp
from jax import lax
from jax.experimental import pallas as pl
from jax.experimental.pallas import tpu as pltpu
from jax.experimental.pallas import tpu_sc as plsc

N_USERS = 50000
N_ITEMS = 50000
D = 32
N = N_USERS + N_ITEMS

NC = 2
NS = 16
LANES = 16

BLK = 128
CHUNK_ROWS = 4
CHUNK_E = BLK * CHUNK_ROWS
NSETS = 3

N_ACC = 100096
ROWS_PER_TILE_OUT = N_ACC // NS


def _sc_body(table, src, dst, val, out, *args, chunks_per_tile):
  bufs = []
  for k in range(NSETS):
    bufs.append(tuple(args[4 * k: 4 * k + 4]))
  acc = args[4 * NSETS]
  sem_g = args[4 * NSETS + 1]
  sem_i = args[4 * NSETS + 2]
  sem_s = args[4 * NSETS + 3: 4 * NSETS + 6]

  c = lax.axis_index("c")
  s = lax.axis_index("s")
  rows_per_tile = chunks_per_tile * CHUNK_ROWS
  tile_row0 = s * rows_per_tile

  rows0 = bufs[0][3]
  def zero_rows(i, _):
    rows0[i, :] = jnp.zeros((LANES,), jnp.float32)
    return 0
  lax.fori_loop(0, CHUNK_E, zero_rows, 0, unroll=4)
  zbase = s * ROWS_PER_TILE_OUT
  for r in range(ROWS_PER_TILE_OUT // CHUNK_E):
    pltpu.sync_copy(rows0, acc.at[pl.ds(zbase + r * CHUNK_E, CHUNK_E)])
  rem = ROWS_PER_TILE_OUT % CHUNK_E
  if rem:
    pltpu.sync_copy(
        rows0.at[pl.ds(0, rem)],
        acc.at[pl.ds(zbase + (ROWS_PER_TILE_OUT // CHUNK_E) * CHUNK_E, rem)])
  plsc.subcore_barrier()

  def load_idx(k, chunk):
    sidx, didx, vv, _ = bufs[k]
    row0 = tile_row0 + chunk * CHUNK_ROWS
    pltpu.async_copy(src.at[pl.ds(row0, CHUNK_ROWS)], sidx, sem_i)
    pltpu.async_copy(dst.at[pl.ds(row0, CHUNK_ROWS)], didx, sem_i)
    pltpu.async_copy(val.at[pl.ds(row0, CHUNK_ROWS)], vv, sem_i)

  def wait_idx(k, chunk):
    sidx, didx, vv, _ = bufs[k]
    row0 = tile_row0 + chunk * CHUNK_ROWS
    pltpu.make_async_copy(src.at[pl.ds(row0, CHUNK_ROWS)], sidx, sem_i).wait()
    pltpu.make_async_copy(dst.at[pl.ds(row0, CHUNK_ROWS)], didx, sem_i).wait()
    pltpu.make_async_copy(val.at[pl.ds(row0, CHUNK_ROWS)], vv, sem_i).wait()

  def fix_idx(k):
    sidx = bufs[k][0]
    def body(i, _):
      for j in range(BLK // LANES):
        sl = pl.ds(j * LANES, LANES)
        sidx[i, sl] = sidx[i, sl] * 2 + c
      return 0
    lax.fori_loop(0, CHUNK_ROWS, body, 0)

  def fire_gather(k):
    sidx, _, _, rows = bufs[k]
    for j in range(CHUNK_ROWS):
      pltpu.async_copy(table.at[sidx.at[j]],
                       rows.at[pl.ds(j * BLK, BLK)], sem_g)

  def wait_gather(k):
    sidx, _, _, rows = bufs[k]
    for j in range(CHUNK_ROWS):
      pltpu.make_async_copy(table.at[sidx.at[j]],
                            rows.at[pl.ds(j * BLK, BLK)], sem_g).wait()

  def scale(k):
    _, _, vv, rows = bufs[k]
    def scale_row(i, _):
      def scale_blk(kb, _):
        base = kb * LANES
        vvec = vv[i, pl.ds(base, LANES)]
        for u in range(LANES):
          e = i * BLK + base + u
          rows[e, :] = rows[e, :] * vvec[u]
        return 0
      lax.fori_loop(0, BLK // LANES, scale_blk, 0)
      return 0
    lax.fori_loop(0, CHUNK_ROWS, scale_row, 0)

  def fire_scatter(k):
    _, didx, _, rows = bufs[k]
    for j in range(CHUNK_ROWS):
      pltpu.async_copy(rows.at[pl.ds(j * BLK, BLK)],
                       acc.at[didx.at[j]], sem_s[k], add=True)

  def drain_scatter(k):
    _, didx, _, rows = bufs[k]
    for j in range(CHUNK_ROWS):
      pltpu.make_async_copy(rows.at[pl.ds(j * BLK, BLK)],
                            acc.at[didx.at[j]], sem_s[k]).wait()

  load_idx(0, 0)
  wait_idx(0, 0)
  fix_idx(0)
  fire_gather(0)

  groups = chunks_per_tile // NSETS

  def group(g, _):
    for p in range(NSETS):
      t = g * NSETS + p
      x = p
      z = (p + 1) % NSETS

      if p < 2:
        @pl.when(g > 0)
        def _():
          drain_scatter(z)
      else:
        drain_scatter(z)

      last = (p == NSETS - 1)

      def refill_front():
        load_idx(z, t + 1)

      def refill_back():
        wait_idx(z, t + 1)
        fix_idx(z)
        fire_gather(z)

      if last:
        @pl.when(g < groups - 1)
        def _():
          refill_front()
      else:
        refill_front()

      wait_gather(x)

      if last:
        @pl.when(g < groups - 1)
        def _():
          refill_back()
      else:
        refill_back()

      scale(x)
      fire_scatter(x)
    return 0

  lax.fori_loop(0, groups, group, 0)
  drain_scatter(NSETS - 2)
  drain_scatter(NSETS - 1)
  plsc.subcore_barrier()

  pltpu.sync_copy(acc.at[pl.ds(s * ROWS_PER_TILE_OUT, ROWS_PER_TILE_OUT)],
                  out.at[c, pl.ds(s * ROWS_PER_TILE_OUT, ROWS_PER_TILE_OUT)])


@jax.jit
def kernel(user_emb, item_emb, norm_adj_indices, norm_adj_values):
  e = norm_adj_indices.shape[1]
  per_unit = NS * CHUNK_E * NSETS
  e_pad = ((e + per_unit - 1) // per_unit) * per_unit
  pad = e_pad - e
  chunks_per_tile = e_pad // (NS * CHUNK_E)

  all_emb = jnp.concatenate([user_emb, item_emb], axis=0)
  table = all_emb.reshape(2 * N, D // 2)

  dst = norm_adj_indices[0].astype(jnp.int32)
  src = norm_adj_indices[1].astype(jnp.int32)
  val = norm_adj_values
  if pad:
    spread = (jnp.arange(pad, dtype=jnp.int32) * 61) % N
    dst = jnp.concatenate([dst, spread])
    src = jnp.concatenate([src, spread])
    val = jnp.concatenate([val, jnp.zeros((pad,), val.dtype)])
  src2 = src.reshape(e_pad // BLK, BLK)
  dst2 = dst.reshape(e_pad // BLK, BLK)
  val2 = val.reshape(e_pad // BLK, BLK)

  set_scratch = []
  for _ in range(NSETS):
    set_scratch += [
        pltpu.VMEM((CHUNK_ROWS, BLK), jnp.int32),
        pltpu.VMEM((CHUNK_ROWS, BLK), jnp.int32),
        pltpu.VMEM((CHUNK_ROWS, BLK), jnp.float32),
        pltpu.VMEM((CHUNK_E, LANES), jnp.float32),
    ]

  mesh = plsc.VectorSubcoreMesh(core_axis_name="c", subcore_axis_name="s")
  out = pl.kernel(
      functools.partial(_sc_body, chunks_per_tile=chunks_per_tile),
      out_type=jax.ShapeDtypeStruct((NC, N_ACC, D // 2), jnp.float32),
      mesh=mesh,
      compiler_params=pltpu.CompilerParams(use_tc_tiling_on_sc=False),
      scratch_types=set_scratch + [
          pltpu.VMEM_SHARED((N_ACC, D // 2), jnp.float32),
          pltpu.SemaphoreType.DMA,
          pltpu.SemaphoreType.DMA,
          pltpu.SemaphoreType.DMA,
          pltpu.SemaphoreType.DMA,
          pltpu.SemaphoreType.DMA,
      ],
  )(table, src2, dst2, val2)

  result = jnp.concatenate([out[0, :N], out[1, :N]], axis=1)
  return (result[:N_USERS], result[N_USERS:])
